# Initial kernel scaffold; baseline (speedup 1.0000x reference)
import jax
import jax.numpy as jnp
from jax.experimental import pallas as pl


def kernel(x, edge_index, W1, b1, W2, b2):
    raise NotImplementedError("write your pallas kernel here")


# speedup vs baseline: 10.9073x; 10.9073x over previous
import functools

import jax
import jax.numpy as jnp
from jax import lax
from jax.experimental import pallas as pl
from jax.experimental.pallas import tpu as pltpu
from jax.experimental.pallas import tpu_sc as plsc

N = 10000
E = 160000
NC = 2
NS = 16
NW = NC * NS
EPW = E // NW
CHUNK = 40
GROUP = 5
NCHUNK = EPW // CHUNK
NGROUP = NCHUNK // GROUP
NP = 10240
SLAB = NP // NS
EPW_P = 5008
PAD = NW * EPW_P - E
BLK = 1000

_MESH = plsc.VectorSubcoreMesh(core_axis_name="c", subcore_axis_name="s",
                               num_cores=NC, num_subcores=NS)
_SC_PARAMS = pltpu.CompilerParams(needs_layout_passes=False,
                                  use_tc_tiling_on_sc=False)


@functools.partial(
    pl.kernel,
    out_type=jax.ShapeDtypeStruct((NW, NP, 1), jnp.float32),
    mesh=_MESH,
    compiler_params=_SC_PARAMS,
    scratch_types=[
        pltpu.VMEM((EPW_P,), jnp.int32),
        pltpu.VMEM((NP, 1), jnp.float32),
    ],
)
def _degree_kernel(col_hbm, zeros_hbm, hist_out, colv, hist):
    cid = lax.axis_index("c")
    sid = lax.axis_index("s")
    wid = cid * NS + sid
    pltpu.sync_copy(col_hbm.at[wid], colv)
    pltpu.sync_copy(zeros_hbm, hist)
    ones = jnp.full((16,), 1.0, jnp.float32)
    zcol = jnp.zeros((16,), jnp.int32)

    def body(j, carry):
        idx = colv[pl.ds(j * 16, 16)]
        plsc.addupdate_scatter(hist, [idx, zcol], ones)
        return carry

    lax.fori_loop(0, EPW_P // 16, body, 0)
    pltpu.sync_copy(hist, hist_out.at[wid])


def _make_propagate(D, ntab):
    scratch = (
        [pltpu.VMEM((NCHUNK, CHUNK), jnp.int32) for _ in range(2)]
        + [pltpu.VMEM((CHUNK, D), jnp.float32) for _ in range(GROUP)]
        + [pltpu.VMEM_SHARED((NP, D), jnp.float32), pltpu.SemaphoreType.DMA]
    )
    out_type = [jax.ShapeDtypeStruct((NC, NP, D), jnp.float32) for _ in range(ntab)]

    @functools.partial(pl.kernel, out_type=out_type, mesh=_MESH,
                       compiler_params=_SC_PARAMS, scratch_types=scratch)
    def prop(*refs):
        tabs = refs[:ntab]
        row_hbm, col_hbm, zeros_hbm = refs[ntab:ntab + 3]
        outs = refs[ntab + 3:2 * ntab + 3]
        rowbuf, colbuf = refs[2 * ntab + 3:2 * ntab + 5]
        gbufs = refs[2 * ntab + 5:2 * ntab + 5 + GROUP]
        acc = refs[-2]
        sem = refs[-1]
        cid = lax.axis_index("c")
        sid = lax.axis_index("s")
        wid = cid * NS + sid
        pltpu.sync_copy(row_hbm.at[wid], rowbuf)
        pltpu.sync_copy(col_hbm.at[wid], colbuf)
        for t in range(ntab):
            pltpu.sync_copy(zeros_hbm, acc.at[pl.ds(sid * SLAB, SLAB)])
            plsc.subcore_barrier()

            def body(g, carry, t=t):
                descs = []
                for b in range(GROUP):
                    descs.append(pltpu.async_copy(
                        tabs[t].at[rowbuf.at[g * GROUP + b]], gbufs[b], sem))
                for b in range(GROUP):
                    descs[b].wait()
                for b in range(GROUP):
                    pltpu.sync_copy(gbufs[b], acc.at[colbuf.at[g * GROUP + b]],
                                    add=True)
                return carry

            lax.fori_loop(0, NGROUP, body, 0)
            plsc.subcore_barrier()
            pltpu.sync_copy(acc.at[pl.ds(sid * SLAB, SLAB)],
                            outs[t].at[cid, pl.ds(sid * SLAB, SLAB)])

    return prop


_prop64_4 = _make_propagate(64, 4)
_prop64_1 = _make_propagate(64, 1)


def _prep(x, hists):
    def body(x_ref, h_ref, xs0_ref, xs1_ref, xs2_ref, xs3_ref, dinv_ref):
        deg = jnp.sum(h_ref[...], axis=0) + 1.0
        dinv = lax.rsqrt(deg)
        xs = x_ref[...] * dinv
        xs0_ref[...] = xs[:, 0:64]
        xs1_ref[...] = xs[:, 64:128]
        xs2_ref[...] = xs[:, 128:192]
        xs3_ref[...] = xs[:, 192:256]
        dinv_ref[...] = dinv

    return pl.pallas_call(
        body,
        grid=(N // BLK,),
        in_specs=[
            pl.BlockSpec((BLK, 256), lambda i: (i, 0)),
            pl.BlockSpec((NW, BLK, 1), lambda i: (0, i, 0)),
        ],
        out_specs=[pl.BlockSpec((BLK, 64), lambda i: (i, 0))] * 4
        + [pl.BlockSpec((BLK, 1), lambda i: (i, 0))],
        out_shape=[jax.ShapeDtypeStruct((N, 64), jnp.float32)] * 4
        + [jax.ShapeDtypeStruct((N, 1), jnp.float32)],
    )(x, hists)


def _mm(ps, xss, dinv, W1, b1, W2):
    def body(p0_ref, p1_ref, p2_ref, p3_ref, xs0_ref, xs1_ref, xs2_ref,
             xs3_ref, dinv_ref, W1_ref, b1_ref, W2_ref, z_ref):
        dv = dinv_ref[...]
        p_refs = (p0_ref, p1_ref, p2_ref, p3_ref)
        xs_refs = (xs0_ref, xs1_ref, xs2_ref, xs3_ref)
        h = b1_ref[...]
        for k in range(4):
            u = (p_refs[k][0] + p_refs[k][1] + xs_refs[k][...]) * dv
            h = h + jnp.dot(u, W1_ref[64 * k:64 * (k + 1), :],
                            preferred_element_type=jnp.float32)
        hs = jnp.maximum(h, 0.0) * dv
        z_ref[...] = jnp.dot(hs, W2_ref[...], preferred_element_type=jnp.float32)

    return pl.pallas_call(
        body,
        grid=(N // BLK,),
        in_specs=[pl.BlockSpec((NC, BLK, 64), lambda i: (0, i, 0))] * 4
        + [pl.BlockSpec((BLK, 64), lambda i: (i, 0))] * 4
        + [
            pl.BlockSpec((BLK, 1), lambda i: (i, 0)),
            pl.BlockSpec((256, 512), lambda i: (0, 0)),
            pl.BlockSpec((1, 512), lambda i: (0, 0)),
            pl.BlockSpec((512, 64), lambda i: (0, 0)),
        ],
        out_specs=pl.BlockSpec((BLK, 64), lambda i: (i, 0)),
        out_shape=jax.ShapeDtypeStruct((N, 64), jnp.float32),
    )(*ps, *xss, dinv, W1, b1, W2)


def _final(q, z, dinv, b2):
    def body(q_ref, z_ref, dinv_ref, b2_ref, out_ref):
        out_ref[...] = ((q_ref[0] + q_ref[1] + z_ref[...]) * dinv_ref[...]
                        + b2_ref[...])

    return pl.pallas_call(
        body,
        grid=(N // BLK,),
        in_specs=[
            pl.BlockSpec((NC, BLK, 64), lambda i: (0, i, 0)),
            pl.BlockSpec((BLK, 64), lambda i: (i, 0)),
            pl.BlockSpec((BLK, 1), lambda i: (i, 0)),
            pl.BlockSpec((1, 64), lambda i: (0, 0)),
        ],
        out_specs=pl.BlockSpec((BLK, 64), lambda i: (i, 0)),
        out_shape=jax.ShapeDtypeStruct((N, 64), jnp.float32),
    )(q, z, dinv, b2)


def kernel(x, edge_index, W1, b1, W2, b2):
    ei = edge_index.astype(jnp.int32)
    row = ei[0].reshape(NW, NCHUNK, CHUNK)
    col = ei[1].reshape(NW, NCHUNK, CHUNK)
    colp = jnp.concatenate(
        [ei[1], jnp.full((PAD,), N, jnp.int32)]).reshape(NW, EPW_P)
    zeros_n = jnp.zeros((NP, 1), jnp.float32)
    z64 = jnp.zeros((SLAB, 64), jnp.float32)

    hists = _degree_kernel(colp, zeros_n)
    *xss, dinv = _prep(x, hists)
    ps = _prop64_4(*xss, row, col, z64)
    z = _mm(ps, xss, dinv, W1, b1[None, :], W2)
    (q,) = _prop64_1(z, row, col, z64)
    out = _final(q, z, dinv, b2[None, :])
    return (out, edge_index)


# speedup vs baseline: 16.4970x; 1.5125x over previous
import functools

import jax
import jax.numpy as jnp
from jax import lax
from jax.experimental import pallas as pl
from jax.experimental.pallas import tpu as pltpu
from jax.experimental.pallas import tpu_sc as plsc

N = 10000
E = 160000
NC = 2
NS = 16
NW = NC * NS
EPW = E // NW
CHUNK = 128
EPT = 5120
GROUP = 5
NCHUNK = EPT // CHUNK
NGROUP = NCHUNK // GROUP
NP = 10240
SLAB = NP // NS
EPW_P = 5008
PAD = NW * EPW_P - E
BLK = 1000

_MESH = plsc.VectorSubcoreMesh(core_axis_name="c", subcore_axis_name="s",
                               num_cores=NC, num_subcores=NS)
_SC_PARAMS = pltpu.CompilerParams(needs_layout_passes=False,
                                  use_tc_tiling_on_sc=False)


@functools.partial(
    pl.kernel,
    out_type=jax.ShapeDtypeStruct((NC, NP, 1), jnp.float32),
    mesh=_MESH,
    compiler_params=_SC_PARAMS,
    scratch_types=[
        pltpu.VMEM((EPW_P,), jnp.int32),
        pltpu.VMEM((NP, 1), jnp.float32),
        pltpu.VMEM((NP // 128, 128), jnp.int32),
        pltpu.VMEM_SHARED((NP, 1), jnp.float32),
        pltpu.SemaphoreType.DMA,
    ],
)
def _degree_kernel(col_hbm, zeros_hbm, iota_hbm, deg_out, colv, hist, iobuf,
                   deg_sh, sem):
    cid = lax.axis_index("c")
    sid = lax.axis_index("s")
    wid = cid * NS + sid
    pltpu.sync_copy(col_hbm.at[wid], colv)
    pltpu.sync_copy(zeros_hbm, hist)
    pltpu.sync_copy(iota_hbm, iobuf)

    @pl.when(sid == 0)
    def _():
        pltpu.sync_copy(zeros_hbm, deg_sh)

    ones = jnp.full((16,), 1.0, jnp.float32)
    zcol = jnp.zeros((16,), jnp.int32)

    def body(j, carry):
        idx = colv[pl.ds(j * 16, 16)]
        plsc.addupdate_scatter(hist, [idx, zcol], ones)
        return carry

    lax.fori_loop(0, EPW_P // 16, body, 0)
    plsc.subcore_barrier()

    def red_body(g, carry):
        descs = []
        for b in range(5):
            j = (g * 5 + b + sid * 5) % (NP // 128)
            descs.append(pltpu.async_copy(
                hist.at[pl.ds(j * 128, 128)], deg_sh.at[iobuf.at[j]], sem,
                add=True))
        for dsc in descs:
            dsc.wait()
        return carry

    lax.fori_loop(0, NP // 128 // 5, red_body, 0)
    plsc.subcore_barrier()

    @pl.when(sid == 0)
    def _():
        pltpu.sync_copy(deg_sh, deg_out.at[cid])


def _make_propagate(D, ntab):
    scratch = (
        [pltpu.VMEM((NCHUNK, CHUNK), jnp.int32) for _ in range(2)]
        + [pltpu.VMEM((CHUNK, D), jnp.float32) for _ in range(GROUP)]
        + [pltpu.VMEM_SHARED((NP, D), jnp.float32), pltpu.SemaphoreType.DMA]
    )
    out_type = [jax.ShapeDtypeStruct((NC, NP, D), jnp.float32) for _ in range(ntab)]

    @functools.partial(pl.kernel, out_type=out_type, mesh=_MESH,
                       compiler_params=_SC_PARAMS, scratch_types=scratch)
    def prop(*refs):
        tabs = refs[:ntab]
        row_hbm, col_hbm, zeros_hbm = refs[ntab:ntab + 3]
        outs = refs[ntab + 3:2 * ntab + 3]
        rowbuf, colbuf = refs[2 * ntab + 3:2 * ntab + 5]
        gbufs = refs[2 * ntab + 5:2 * ntab + 5 + GROUP]
        acc = refs[-2]
        sem = refs[-1]
        cid = lax.axis_index("c")
        sid = lax.axis_index("s")
        wid = cid * NS + sid
        pltpu.sync_copy(row_hbm.at[wid], rowbuf)
        pltpu.sync_copy(col_hbm.at[wid], colbuf)
        for t in range(ntab):
            pltpu.sync_copy(zeros_hbm, acc.at[pl.ds(sid * SLAB, SLAB)])
            plsc.subcore_barrier()

            def body(g, carry, t=t):
                descs = []
                for b in range(GROUP):
                    descs.append(pltpu.async_copy(
                        tabs[t].at[rowbuf.at[g * GROUP + b]], gbufs[b], sem))
                for b in range(GROUP):
                    descs[b].wait()
                for b in range(GROUP):
                    pltpu.sync_copy(gbufs[b], acc.at[colbuf.at[g * GROUP + b]],
                                    add=True)
                return carry

            lax.fori_loop(0, NGROUP, body, 0)
            plsc.subcore_barrier()
            pltpu.sync_copy(acc.at[pl.ds(sid * SLAB, SLAB)],
                            outs[t].at[cid, pl.ds(sid * SLAB, SLAB)])

    return prop


_prop64_4 = _make_propagate(64, 4)
_prop64_1 = _make_propagate(64, 1)


def _prep(x, hists):
    def body(x_ref, h_ref, xs0_ref, xs1_ref, xs2_ref, xs3_ref, dinv_ref):
        deg = jnp.sum(h_ref[...], axis=0) + 1.0
        dinv = lax.rsqrt(deg)
        xs = x_ref[...] * dinv
        xs0_ref[...] = xs[:, 0:64]
        xs1_ref[...] = xs[:, 64:128]
        xs2_ref[...] = xs[:, 128:192]
        xs3_ref[...] = xs[:, 192:256]
        dinv_ref[...] = dinv

    return pl.pallas_call(
        body,
        grid=(N // BLK,),
        in_specs=[
            pl.BlockSpec((BLK, 256), lambda i: (i, 0)),
            pl.BlockSpec((NC, BLK, 1), lambda i: (0, i, 0)),
        ],
        out_specs=[pl.BlockSpec((BLK, 64), lambda i: (i, 0))] * 4
        + [pl.BlockSpec((BLK, 1), lambda i: (i, 0))],
        out_shape=[jax.ShapeDtypeStruct((N, 64), jnp.float32)] * 4
        + [jax.ShapeDtypeStruct((N, 1), jnp.float32)],
    )(x, hists)


def _mm(ps, xss, dinv, W1, b1, W2):
    def body(p0_ref, p1_ref, p2_ref, p3_ref, xs0_ref, xs1_ref, xs2_ref,
             xs3_ref, dinv_ref, W1_ref, b1_ref, W2_ref, z_ref):
        dv = dinv_ref[...]
        p_refs = (p0_ref, p1_ref, p2_ref, p3_ref)
        xs_refs = (xs0_ref, xs1_ref, xs2_ref, xs3_ref)
        h = b1_ref[...]
        for k in range(4):
            u = (p_refs[k][0] + p_refs[k][1] + xs_refs[k][...]) * dv
            h = h + jnp.dot(u, W1_ref[64 * k:64 * (k + 1), :],
                            preferred_element_type=jnp.float32)
        hs = jnp.maximum(h, 0.0) * dv
        z_ref[...] = jnp.dot(hs, W2_ref[...], preferred_element_type=jnp.float32)

    return pl.pallas_call(
        body,
        grid=(N // BLK,),
        in_specs=[pl.BlockSpec((NC, BLK, 64), lambda i: (0, i, 0))] * 4
        + [pl.BlockSpec((BLK, 64), lambda i: (i, 0))] * 4
        + [
            pl.BlockSpec((BLK, 1), lambda i: (i, 0)),
            pl.BlockSpec((256, 512), lambda i: (0, 0)),
            pl.BlockSpec((1, 512), lambda i: (0, 0)),
            pl.BlockSpec((512, 64), lambda i: (0, 0)),
        ],
        out_specs=pl.BlockSpec((BLK, 64), lambda i: (i, 0)),
        out_shape=jax.ShapeDtypeStruct((N, 64), jnp.float32),
    )(*ps, *xss, dinv, W1, b1, W2)


def _final(q, z, dinv, b2):
    def body(q_ref, z_ref, dinv_ref, b2_ref, out_ref):
        out_ref[...] = ((q_ref[0] + q_ref[1] + z_ref[...]) * dinv_ref[...]
                        + b2_ref[...])

    return pl.pallas_call(
        body,
        grid=(N // BLK,),
        in_specs=[
            pl.BlockSpec((NC, BLK, 64), lambda i: (0, i, 0)),
            pl.BlockSpec((BLK, 64), lambda i: (i, 0)),
            pl.BlockSpec((BLK, 1), lambda i: (i, 0)),
            pl.BlockSpec((1, 64), lambda i: (0, 0)),
        ],
        out_specs=pl.BlockSpec((BLK, 64), lambda i: (i, 0)),
        out_shape=jax.ShapeDtypeStruct((N, 64), jnp.float32),
    )(q, z, dinv, b2)


def kernel(x, edge_index, W1, b1, W2, b2):
    ei = edge_index.astype(jnp.int32)
    npad = NW * (EPT - EPW)
    fill_r = (jnp.arange(npad, dtype=jnp.int32) % N).reshape(NW, EPT - EPW)
    fill_c = (N + jnp.arange(npad, dtype=jnp.int32) % (NP - N)).reshape(
        NW, EPT - EPW)
    row = jnp.concatenate([ei[0].reshape(NW, EPW), fill_r],
                          axis=1).reshape(NW, NCHUNK, CHUNK)
    col = jnp.concatenate([ei[1].reshape(NW, EPW), fill_c],
                          axis=1).reshape(NW, NCHUNK, CHUNK)
    colp = jnp.concatenate(
        [ei[1], jnp.full((PAD,), N, jnp.int32)]).reshape(NW, EPW_P)
    zeros_n = jnp.zeros((NP, 1), jnp.float32)
    z64 = jnp.zeros((SLAB, 64), jnp.float32)
    iota_np = jnp.arange(NP, dtype=jnp.int32).reshape(NP // 128, 128)

    hists = _degree_kernel(colp, zeros_n, iota_np)
    *xss, dinv = _prep(x, hists)
    ps = _prop64_4(*xss, row, col, z64)
    z = _mm(ps, xss, dinv, W1, b1[None, :], W2)
    (q,) = _prop64_1(z, row, col, z64)
    out = _final(q, z, dinv, b2[None, :])
    return (out, edge_index)
